# baseline (speedup 1.0000x reference)
import functools

import jax
import jax.numpy as jnp
from jax import lax
from jax.experimental import pallas as pl
from jax.experimental.pallas import tpu as pltpu
from jax.experimental.pallas import tpu_sc as plsc

_NUM_TOKENS = 512
_DIM = 1024
_NUM_CHAPTERS = 16
_T = 32
_BATCH = 2048
_K = 2
_NPAIRS = _BATCH * _K
_NC = 2
_NS = 16
_NW = _NC * _NS
_SC_PAIRS = 2048
_WINDOW = 8


def _sc_gather_kernel(mem_hbm, cidx_hbm, out_hbm, aidx_hbm,
                      bank, cidx_v, aidx_v, out_sem, in_sem):
    ppw = _SC_PAIRS // _NW
    cid = lax.axis_index("c")
    sid = lax.axis_index("s")
    wid = sid * _NC + cid
    base = wid * ppw

    pltpu.async_copy(cidx_hbm.at[pl.ds(base, ppw)], cidx_v, in_sem)

    @pl.when(sid == 0)
    def _():
        pltpu.sync_copy(mem_hbm, bank)

    plsc.subcore_barrier()
    pltpu.make_async_copy(cidx_hbm.at[pl.ds(base, ppw)], cidx_v,
                          in_sem).wait()

    iota = lax.broadcasted_iota(jnp.int32, (16,), 0)
    descs = []
    for g in range(ppw // 16):
        cvec = cidx_v[pl.ds(g * 16, 16)]
        for l in range(16):
            p = g * 16 + l
            c = cvec[l]
            row0 = c * _T
            lo = row0 + iota
            aidx_v[p, pl.ds(0, 16)] = lo
            aidx_v[p, pl.ds(16, 16)] = lo + 16
            d = pltpu.async_copy(bank.at[pl.ds(row0, _T)],
                                 out_hbm.at[base + p], out_sem)
            descs.append(d)
            if len(descs) > _WINDOW:
                descs.pop(0).wait()

    pltpu.sync_copy(aidx_v, aidx_hbm.at[pl.ds(base, ppw)])
    for d in descs:
        d.wait()


def _sc_call(memory, cidx_sc):
    mesh = plsc.VectorSubcoreMesh(core_axis_name="c", subcore_axis_name="s")
    ppw = _SC_PAIRS // _NW
    return pl.kernel(
        _sc_gather_kernel,
        out_type=(
            jax.ShapeDtypeStruct((_SC_PAIRS, _T, _DIM), jnp.float32),
            jax.ShapeDtypeStruct((_SC_PAIRS, _T), jnp.int32),
        ),
        mesh=mesh,
        scratch_types=[
            pltpu.VMEM_SHARED((_NUM_TOKENS, _DIM), jnp.float32),
            pltpu.VMEM((ppw,), jnp.int32),
            pltpu.VMEM((ppw, _T), jnp.int32),
            pltpu.SemaphoreType.DMA,
            pltpu.SemaphoreType.DMA,
        ],
    )(memory, cidx_sc)


def _tc_body(cidx_ref, mem_ref, out_ref, aidx_ref):
    i = pl.program_id(0)
    c = cidx_ref[i]
    out_ref[0] = mem_ref[pl.ds(c * _T, _T), :]
    aidx_ref[0, 0] = c * _T + lax.broadcasted_iota(jnp.int32, (_T,), 0)


def _tc_call(memory, cidx_tc):
    n = cidx_tc.shape[0]
    grid_spec = pltpu.PrefetchScalarGridSpec(
        num_scalar_prefetch=1,
        grid=(n,),
        in_specs=[
            pl.BlockSpec((_NUM_TOKENS, _DIM), lambda i, s: (0, 0)),
        ],
        out_specs=[
            pl.BlockSpec((1, _T, _DIM), lambda i, s: (i, 0, 0)),
            pl.BlockSpec((1, 1, _T), lambda i, s: (i, 0, 0)),
        ],
    )
    tc_g, tc_i = pl.pallas_call(
        _tc_body,
        grid_spec=grid_spec,
        out_shape=(
            jax.ShapeDtypeStruct((n, _T, _DIM), jnp.float32),
            jax.ShapeDtypeStruct((n, 1, _T), jnp.int32),
        ),
    )(cidx_tc, memory)
    return tc_g, tc_i.reshape(n, _T)


def kernel(memory, chapter_indices):
    cidx_flat = chapter_indices.reshape(_NPAIRS).astype(jnp.int32)
    gathered, aidx = _tc_call(memory, cidx_flat)
    return (gathered.reshape(_BATCH, _K * _T, _DIM),
            aidx.reshape(_BATCH, _K * _T).astype(chapter_indices.dtype))


# speedup vs baseline: 6.9132x; 6.9132x over previous
import jax
import jax.numpy as jnp
from jax import lax
from jax.experimental import pallas as pl
from jax.experimental.pallas import tpu as pltpu
from jax.experimental.pallas import tpu_sc as plsc

_NUM_TOKENS = 512
_DIM = 1024
_NUM_CHAPTERS = 16
_T = 32
_BATCH = 2048
_K = 2
_NPAIRS = _BATCH * _K
_NC = 2
_NS = 16
_HALF = _NPAIRS // _NC
_PPT = _HALF // _NS


def _sc_gather_kernel(mem_hbm, cidx_hbm, out_hbm, aidx_hbm,
                      mychap, cidx_v, aidx_v, out_sem, in_sem, chap_sem):
    cid = lax.axis_index("c")
    sid = lax.axis_index("s")
    half = cid * _HALF

    pltpu.async_copy(cidx_hbm.at[pl.ds(half, _HALF)], cidx_v, in_sem)
    pltpu.async_copy(mem_hbm.at[pl.ds(sid * _T, _T)], mychap, chap_sem)
    pltpu.make_async_copy(cidx_hbm.at[pl.ds(half, _HALF)], cidx_v,
                          in_sem).wait()

    iota = lax.broadcasted_iota(jnp.int32, (16,), 0)
    for g in range(_PPT // 16):
        cvec = cidx_v[pl.ds(sid * _PPT + g * 16, 16)]
        for l in range(16):
            p = g * 16 + l
            row0 = cvec[l] * _T
            lo = row0 + iota
            aidx_v[p, pl.ds(0, 16)] = lo
            aidx_v[p, pl.ds(16, 16)] = lo + 16
    pltpu.sync_copy(aidx_v, aidx_hbm.at[pl.ds(half + sid * _PPT, _PPT)])

    pltpu.make_async_copy(mem_hbm.at[pl.ds(sid * _T, _T)], mychap,
                          chap_sem).wait()

    def scan_body(g, cnt):
        vec = cidx_v[pl.ds(g * 16, 16)]
        for l in range(16):
            c = vec[l]
            hit = c == sid

            @pl.when(hit)
            def _():
                pltpu.async_copy(mychap, out_hbm.at[half + g * 16 + l],
                                 out_sem)

            cnt = jnp.where(hit, cnt + 1, cnt)
        return cnt

    n_served = lax.fori_loop(0, _HALF // 16, scan_body, jnp.int32(0))

    def drain_body(i, carry):
        pltpu.make_async_copy(mem_hbm.at[pl.ds(0, _T)], mychap,
                              out_sem).wait()
        return carry

    lax.fori_loop(0, n_served, drain_body, jnp.int32(0))


def kernel(memory, chapter_indices):
    cidx_flat = chapter_indices.reshape(_NPAIRS).astype(jnp.int32)
    mesh = plsc.VectorSubcoreMesh(core_axis_name="c", subcore_axis_name="s")
    gathered, aidx = pl.kernel(
        _sc_gather_kernel,
        out_type=(
            jax.ShapeDtypeStruct((_NPAIRS, _T, _DIM), jnp.float32),
            jax.ShapeDtypeStruct((_NPAIRS, _T), jnp.int32),
        ),
        mesh=mesh,
        scratch_types=[
            pltpu.VMEM((_T, _DIM), jnp.float32),
            pltpu.VMEM((_HALF,), jnp.int32),
            pltpu.VMEM((_PPT, _T), jnp.int32),
            pltpu.SemaphoreType.DMA,
            pltpu.SemaphoreType.DMA,
            pltpu.SemaphoreType.DMA,
        ],
    )(memory, cidx_flat)
    return (gathered.reshape(_BATCH, _K * _T, _DIM),
            aidx.reshape(_BATCH, _K * _T).astype(chapter_indices.dtype))
